# baseline (speedup 1.0000x reference)
import functools

import jax
import jax.numpy as jnp
from jax.experimental import pallas as pl
from jax.experimental.pallas import tpu as pltpu


def _moe_dense_body(x_ref, wg_ref, w1_ref, w2_ref, out_ref, logits_ref,
                    acc_ref, comb_ref, *, n_e, n_f):
    e = pl.program_id(1)
    f = pl.program_id(2)
    tm = x_ref.shape[0]
    n_exp = wg_ref.shape[0]

    @pl.when((e == 0) & (f == 0))
    def _router():
        xt = x_ref[...]
        logits = jax.lax.dot_general(
            xt, wg_ref[...], (((1,), (1,)), ((), ())),
            preferred_element_type=jnp.float32)
        logits_ref[...] = logits
        ids = jax.lax.broadcasted_iota(jnp.int32, (tm, n_exp), 1)
        m1 = jnp.max(logits, axis=1, keepdims=True)
        i1 = jnp.min(jnp.where(logits == m1, ids, n_exp), axis=1, keepdims=True)
        masked = jnp.where(ids == i1, -jnp.inf, logits)
        m2 = jnp.max(masked, axis=1, keepdims=True)
        i2 = jnp.min(jnp.where(masked == m2, ids, n_exp), axis=1, keepdims=True)
        z = jnp.exp(m2 - m1)
        w_hi = 1.0 / (1.0 + z)
        w_lo = z / (1.0 + z)
        comb_ref[...] = (jnp.where(ids == i1, w_hi, 0.0)
                         + jnp.where(ids == i2, w_lo, 0.0))
        acc_ref[...] = jnp.zeros_like(acc_ref)

    xt = x_ref[...]
    h = jax.lax.dot_general(
        xt, w1_ref[0], (((1,), (1,)), ((), ())),
        preferred_element_type=jnp.float32)
    h = jnp.maximum(h, 0.0)
    ids = jax.lax.broadcasted_iota(jnp.int32, (tm, n_exp), 1)
    c = jnp.sum(jnp.where(ids == e, comb_ref[...], 0.0), axis=1,
                keepdims=True)
    hw = h * c
    acc_ref[...] += jax.lax.dot_general(
        hw, w2_ref[0], (((1,), (1,)), ((), ())),
        preferred_element_type=jnp.float32)

    @pl.when((e == n_e - 1) & (f == n_f - 1))
    def _emit():
        out_ref[...] = acc_ref[...]


def kernel(x, Wg, W1, W2):
    b, t, hdim = x.shape
    n_exp, ff, _ = W1.shape
    bt = b * t
    tm = min(512, bt)
    fc = min(1024, ff)
    n_t, n_f = bt // tm, ff // fc

    xf = x.reshape(bt, hdim)
    body = functools.partial(_moe_dense_body, n_e=n_exp, n_f=n_f)
    out, logits = pl.pallas_call(
        body,
        grid=(n_t, n_exp, n_f),
        in_specs=[
            pl.BlockSpec((tm, hdim), lambda ti, e, f: (ti, 0)),
            pl.BlockSpec((n_exp, hdim), lambda ti, e, f: (0, 0)),
            pl.BlockSpec((1, fc, hdim), lambda ti, e, f: (e, f, 0)),
            pl.BlockSpec((1, hdim, fc), lambda ti, e, f: (e, 0, f)),
        ],
        out_specs=[
            pl.BlockSpec((tm, hdim), lambda ti, e, f: (ti, 0)),
            pl.BlockSpec((tm, n_exp), lambda ti, e, f: (ti, 0)),
        ],
        out_shape=[
            jax.ShapeDtypeStruct((bt, hdim), jnp.float32),
            jax.ShapeDtypeStruct((bt, n_exp), jnp.float32),
        ],
        scratch_shapes=[
            pltpu.VMEM((tm, hdim), jnp.float32),
            pltpu.VMEM((tm, n_exp), jnp.float32),
        ],
        compiler_params=pltpu.CompilerParams(
            dimension_semantics=("parallel", "arbitrary", "arbitrary")),
    )(xf, Wg, W1, W2)
    return out.reshape(b, t, hdim), logits.reshape(b, t, n_exp)
